# baseline (speedup 1.0000x reference)
import functools

import numpy as np
import jax
import jax.numpy as jnp
from jax import lax
from jax.experimental import pallas as pl
from jax.experimental.pallas import tpu as pltpu
from jax.experimental.pallas import tpu_sc as plsc

_NT = 4096
_NSEG = 512
_NB = 4
_D = 1024

_NC = 2
_NS = 16
_NW = _NC * _NS

_B_TOTAL = _NB * _NSEG
_B_PER_W = _B_TOTAL // _NW


def _segment_rows() -> np.ndarray:
    t = np.linspace(1, _NT, _NSEG + 1)
    t = np.asarray([int(round(x)) - 1 for x in t][:-1], dtype=np.int32)
    rows = t[None, :] + (np.arange(_NB, dtype=np.int32) * _NT)[:, None]
    return rows.reshape(-1)


_ROW_IDS = _segment_rows()

_mesh = plsc.VectorSubcoreMesh(core_axis_name="c", subcore_axis_name="s")


_NCHUNK = 4
_CH = _B_PER_W // _NCHUNK


@functools.partial(
    pl.kernel,
    mesh=_mesh,
    out_type=jax.ShapeDtypeStruct((_B_TOTAL, _D), jnp.float32),
    scratch_types=[
        pltpu.VMEM((_B_PER_W,), jnp.int32),
        pltpu.VMEM((_B_PER_W, _D), jnp.float32),
        pltpu.SemaphoreType.DMA,
        pltpu.SemaphoreType.DMA,
    ],
)
def _gather_rows(table_hbm, idx_hbm, out_hbm, idx_v, rows_v, gsem, ssem):
    wid = lax.axis_index("s") * _NC + lax.axis_index("c")
    base = wid * _B_PER_W
    pltpu.sync_copy(idx_hbm.at[pl.ds(base, _B_PER_W)], idx_v)
    h = _B_PER_W // 2
    g0 = pltpu.async_copy(
        table_hbm.at[idx_v.at[pl.ds(0, h)]], rows_v.at[pl.ds(0, h)], gsem)
    g1 = pltpu.async_copy(
        table_hbm.at[idx_v.at[pl.ds(h, h)]], rows_v.at[pl.ds(h, h)], gsem)
    g0.wait()
    s0 = pltpu.async_copy(
        rows_v.at[pl.ds(0, h)], out_hbm.at[pl.ds(base, h)], ssem)
    g1.wait()
    s1 = pltpu.async_copy(
        rows_v.at[pl.ds(h, h)], out_hbm.at[pl.ds(base + h, h)], ssem)
    s0.wait()
    s1.wait()


def kernel(inp, n_segments):
    del n_segments
    nb, nt, d = inp.shape
    table = inp.reshape(nb * nt, d)
    idx = jnp.asarray(_ROW_IDS)
    out = _gather_rows(table, idx)
    return out.reshape(nb, _NSEG, d)


# speedup vs baseline: 1.0312x; 1.0312x over previous
import functools

import jax
import jax.numpy as jnp
from jax import lax
from jax.experimental import pallas as pl
from jax.experimental.pallas import tpu as pltpu
from jax.experimental.pallas import tpu_sc as plsc

_NT = 4096
_NSEG = 512
_NB = 4
_D = 1024

_NC = 2
_NS = 16
_NW = _NC * _NS
_LANES = 16

_B_TOTAL = _NB * _NSEG
_B_PER_W = _B_TOTAL // _NW

_mesh = plsc.VectorSubcoreMesh(core_axis_name="c", subcore_axis_name="s")


@functools.partial(
    pl.kernel,
    mesh=_mesh,
    out_type=jax.ShapeDtypeStruct((_B_TOTAL, _D), jnp.float32),
    scratch_types=[
        pltpu.VMEM((_B_PER_W,), jnp.int32),
        pltpu.VMEM((_B_PER_W, _D), jnp.float32),
        pltpu.SemaphoreType.DMA,
    ],
)
def _gather_rows(table_hbm, out_hbm, idx_v, rows_v, sem):
    wid = lax.axis_index("s") * _NC + lax.axis_index("c")
    base = wid * _B_PER_W
    for c in range(_B_PER_W // _LANES):
        r = base + c * _LANES + lax.iota(jnp.int32, _LANES)
        b = lax.shift_right_logical(r, 9)
        s = lax.bitwise_and(r, 511)
        t_raw = lax.shift_right_logical(s * 4095 + 256, 9)
        t = t_raw - jnp.where(s == 256, 1, 0).astype(jnp.int32)
        idx_v[pl.ds(c * _LANES, _LANES)] = lax.shift_left(b, 12) + t
    pltpu.async_copy(table_hbm.at[idx_v], rows_v, sem).wait()
    pltpu.sync_copy(rows_v, out_hbm.at[pl.ds(base, _B_PER_W)])


def kernel(inp, n_segments):
    del n_segments
    nb, nt, d = inp.shape
    table = inp.reshape(nb * nt, d)
    out = _gather_rows(table)
    return out.reshape(nb, _NSEG, d)
